# baseline (speedup 1.0000x reference)
import functools
import math

import jax
import jax.numpy as jnp
from jax.experimental import pallas as pl
from jax.experimental.pallas import tpu as pltpu

_PADDING_IDX = 0
_BLK_ROWS = 1024


_INIT_ROWS = 16


def _table_kernel(o_ref, ds_ref, dc_ref, *, lo, scale, half_dim, blk_rows):
    i = pl.program_id(0)
    jj = pl.program_id(1)

    col1 = jax.lax.broadcasted_iota(jnp.int32, (1, half_dim), 1).astype(
        jnp.float32
    )
    f1 = jnp.exp(col1 * jnp.float32(-scale))

    @pl.when(jnp.logical_and(i == 0, jj == 0))
    def _init_delta():
        sub = min(_INIT_ROWS, blk_rows)
        col = jax.lax.broadcasted_iota(jnp.int32, (sub, half_dim), 1).astype(
            jnp.float32
        )
        freqs = jnp.exp(col * jnp.float32(-scale))
        k = jax.lax.broadcasted_iota(jnp.int32, (sub, half_dim), 0).astype(
            jnp.float32
        )
        d = k * freqs
        ds_ref[:sub] = jnp.sin(d)
        dc_ref[:sub] = jnp.cos(d)
        n = sub
        while n < blk_rows:
            rs = jnp.sin(jnp.float32(n) * f1)
            rc = jnp.cos(jnp.float32(n) * f1)
            s_lo = ds_ref[:n]
            c_lo = dc_ref[:n]
            ds_ref[n : 2 * n] = rs * c_lo + rc * s_lo
            dc_ref[n : 2 * n] = rc * c_lo - rs * s_lo
            n *= 2
    base_arg = (i * blk_rows + lo).astype(jnp.float32) * f1
    bs = jnp.sin(base_arg)
    bc = jnp.cos(base_arg)
    ds = ds_ref[:]
    dc = dc_ref[:]

    @pl.when(jj == 0)
    def _sin_half():
        o_ref[:] = bs * dc + bc * ds

    @pl.when(jj == 1)
    def _cos_half():
        o_ref[:] = bc * dc - bs * ds

    @pl.when(i == (_PADDING_IDX // blk_rows))
    def _zero_padding_row():
        o_ref[_PADDING_IDX % blk_rows, :] = jnp.zeros((half_dim,), jnp.float32)


def kernel(input, weights):
    bsz, seq_len = input.shape
    embed_dim = weights.shape[1]
    half_dim = embed_dim // 2
    num_embeddings = 2 * seq_len
    lo = -(num_embeddings // 2)
    scale = math.log(10000.0) / (half_dim - 1)

    blk = min(_BLK_ROWS, num_embeddings)
    grid = num_embeddings // blk

    body = functools.partial(
        _table_kernel, lo=lo, scale=scale, half_dim=half_dim, blk_rows=blk
    )
    return pl.pallas_call(
        body,
        out_shape=jax.ShapeDtypeStruct((num_embeddings, embed_dim), jnp.float32),
        grid=(grid, 2),
        out_specs=pl.BlockSpec((blk, half_dim), lambda i, j: (i, j)),
        scratch_shapes=[
            pltpu.VMEM((blk, half_dim), jnp.float32),
            pltpu.VMEM((blk, half_dim), jnp.float32),
        ],
    )()


# speedup vs baseline: 1.3711x; 1.3711x over previous
import functools
import math

import jax
import jax.numpy as jnp
from jax.experimental import pallas as pl
from jax.experimental.pallas import tpu as pltpu

_PADDING_IDX = 0
_BLK_ROWS = 1024


_INIT_ROWS = 16


def _table_kernel(o_ref, ds_ref, dc_ref, *, lo, scale, half_dim, blk_rows):
    i = pl.program_id(0)

    col1 = jax.lax.broadcasted_iota(jnp.int32, (1, half_dim), 1).astype(
        jnp.float32
    )
    f1 = jnp.exp(col1 * jnp.float32(-scale))

    @pl.when(i == 0)
    def _init_delta():
        sub = min(_INIT_ROWS, blk_rows)
        col = jax.lax.broadcasted_iota(jnp.int32, (sub, half_dim), 1).astype(
            jnp.float32
        )
        freqs = jnp.exp(col * jnp.float32(-scale))
        k = jax.lax.broadcasted_iota(jnp.int32, (sub, half_dim), 0).astype(
            jnp.float32
        )
        d = k * freqs
        ds_ref[:sub] = jnp.sin(d)
        dc_ref[:sub] = jnp.cos(d)
        n = sub
        while n < blk_rows:
            rs = jnp.sin(jnp.float32(n) * f1)
            rc = jnp.cos(jnp.float32(n) * f1)
            s_lo = ds_ref[:n]
            c_lo = dc_ref[:n]
            ds_ref[n : 2 * n] = rs * c_lo + rc * s_lo
            dc_ref[n : 2 * n] = rc * c_lo - rs * s_lo
            n *= 2
    base_arg = (i * blk_rows + lo).astype(jnp.float32) * f1
    bs = jnp.sin(base_arg)
    bc = jnp.cos(base_arg)
    ds = ds_ref[:]
    dc = dc_ref[:]
    o_ref[:, :half_dim] = bs * dc + bc * ds
    o_ref[:, half_dim:] = bc * dc - bs * ds

    @pl.when(i == (_PADDING_IDX // blk_rows))
    def _zero_padding_row():
        o_ref[_PADDING_IDX % blk_rows, :] = jnp.zeros(
            (2 * half_dim,), jnp.float32
        )


def kernel(input, weights):
    bsz, seq_len = input.shape
    embed_dim = weights.shape[1]
    half_dim = embed_dim // 2
    num_embeddings = 2 * seq_len
    lo = -(num_embeddings // 2)
    scale = math.log(10000.0) / (half_dim - 1)

    blk = min(_BLK_ROWS, num_embeddings)
    grid = num_embeddings // blk

    body = functools.partial(
        _table_kernel, lo=lo, scale=scale, half_dim=half_dim, blk_rows=blk
    )
    return pl.pallas_call(
        body,
        out_shape=jax.ShapeDtypeStruct((num_embeddings, embed_dim), jnp.float32),
        grid=(grid,),
        out_specs=pl.BlockSpec((blk, embed_dim), lambda i: (i, 0)),
        scratch_shapes=[
            pltpu.VMEM((blk, half_dim), jnp.float32),
            pltpu.VMEM((blk, half_dim), jnp.float32),
        ],
    )()
